# baseline (speedup 1.0000x reference)
import functools

import jax
import jax.numpy as jnp
from jax import lax
from jax.experimental import pallas as pl
from jax.experimental.pallas import tpu as pltpu
from jax.experimental.pallas import tpu_sc as plsc

_DIM = 32
_B = 16384
_NC = 2
_NS = 16
_NW = _NC * _NS

_NCLS = 11
_NREL = 2
_NG = _NCLS + _NREL
_CLS_PER_W = _NCLS * _B // _NW
_REL_PER_W = _NREL * _B // _NW
_CH_CLS = 352
_CH_REL = 256
_NCH_CLS = _CLS_PER_W // _CH_CLS
_NCH_REL = _REL_PER_W // _CH_REL


def _tr_body(a_ref, out_ref):
    a = a_ref[...]
    d, w = a.shape
    blk = out_ref.shape[0]
    a2 = jnp.concatenate(
        [a[:, m * blk:(m + 1) * blk] for m in range(w // blk)], axis=0)
    eye = (lax.broadcasted_iota(jnp.int32, (128, 128), 0)
           == lax.broadcasted_iota(jnp.int32, (128, 128), 1)
           ).astype(jnp.float32)
    out_ref[...] = lax.dot_general(a2, eye, (((0,), (0,)), ((), ())),
                                   preferred_element_type=jnp.float32)


def _pack_table(emb, win):
    n, d = emb.shape
    v = emb.T
    blk = win * d // 128
    grid = (n + win - 1) // win
    return pl.pallas_call(
        _tr_body,
        grid=(grid,),
        in_specs=[pl.BlockSpec((d, win), lambda i: (0, i))],
        out_specs=pl.BlockSpec((blk, 128), lambda i: (i, 0)),
        out_shape=jax.ShapeDtypeStruct((grid * blk, 128), jnp.float32),
    )(v)


@functools.cache
def _make_sc_gather(total_rows, chunk):
    per_w = total_rows // _NW
    nch = per_w // chunk

    @functools.partial(
        pl.kernel,
        mesh=plsc.VectorSubcoreMesh(core_axis_name="c", subcore_axis_name="s"),
        out_type=jax.ShapeDtypeStruct((total_rows, 128), jnp.float32),
        scratch_types=[
            pltpu.VMEM((chunk,), jnp.int32),
            pltpu.VMEM((chunk,), jnp.int32),
            pltpu.VMEM((chunk, 128), jnp.float32),
            pltpu.VMEM((chunk, 128), jnp.float32),
            pltpu.SemaphoreType.DMA,
            pltpu.SemaphoreType.DMA,
        ],
    )
    def _sc_gather(idx_h, tab_h, out_h,
                   idx_a, idx_b, rows_a, rows_b, sem_a, sem_b):
        wid = lax.axis_index("s") * _NC + lax.axis_index("c")
        base = wid * per_w
        bufs = [(idx_a, rows_a, sem_a), (idx_b, rows_b, sem_b)]
        handles = [None, None]

        def start(k):
            i_v, r_v, s_v = bufs[k % 2]
            pltpu.sync_copy(idx_h.at[pl.ds(base + k * chunk, chunk)], i_v)
            handles[k % 2] = pltpu.async_copy(tab_h.at[i_v], r_v, s_v)

        start(0)
        for k in range(nch):
            if k + 1 < nch:
                start(k + 1)
            _, r_v, _ = bufs[k % 2]
            handles[k % 2].wait()
            pltpu.sync_copy(r_v, out_h.at[pl.ds(base + k * chunk, chunk)])

    return _sc_gather


_BB = 2048

_EMITS = [
    (64, {0: 0, 1: 1}), (64, {0: 2}), (64, {0: 3}),
    (64, {0: 4}), (64, {1: 5}), (64, {0: 6}), (64, {0: 7}), (64, {0: 8}),
    (128, {0: 9, 1: 10}), (128, {0: 11, 2: 12}),
    (128, {0: 13, 1: 14}), (128, {0: 15, 2: 16}),
    (64, {1: 17}), (64, {0: 18}), (64, {0: 19}),
]


@functools.cache
def _route_matrix():
    import numpy as np
    rows = []
    for w, mp in _EMITS:
        m = np.zeros((w, 128), np.float32)
        for b, c in mp.items():
            m[b * 32:(b + 1) * 32, c] = 1.0
        rows.append(m)
    return np.concatenate(rows)


def _tc_loss_body(crows_ref, rrows_ref, *rest):
    idx_refs = rest[:_NG]
    w4_ref, m_ref, out_ref = rest[_NG], rest[_NG + 1], rest[_NG + 2]
    d = _DIM

    def sel(k):
        g = crows_ref[k]
        b = ((idx_refs[k][...] & 1).reshape(_BB, 1)) != 0
        return jnp.where(b, g[:, 64:128], g[:, 0:64])

    def halves(k):
        row = sel(k)
        return row[:, :d], row[:, d:]

    def rel(k):
        g = rrows_ref[k]
        s = idx_refs[_NCLS + k][...]
        b0 = ((s & 1).reshape(_BB, 1)) != 0
        b1 = ((s & 2).reshape(_BB, 1)) != 0
        w01 = jnp.where(b0, g[:, 32:64], g[:, 0:32])
        w23 = jnp.where(b0, g[:, 96:128], g[:, 64:96])
        return jnp.where(b1, w23, w01)

    def mm4(r1, r2):
        return jnp.dot(jnp.concatenate([r1, r2], axis=1), w4_ref[...],
                       preferred_element_type=jnp.float32)

    s2 = jnp.zeros((_BB, 128), jnp.float32)
    moff = [0]

    def emit(tb):
        nonlocal s2
        w = tb.shape[1]
        m = m_ref[moff[0]:moff[0] + w, :]
        moff[0] += w
        t = jnp.maximum(tb, 0.0)
        s2 = s2 + jnp.dot(t * t, m, preferred_element_type=jnp.float32)

    sg = jnp.where(lax.broadcasted_iota(jnp.int32, (1, 64), 1) < 32,
                   1.0, -1.0)
    sg128 = jnp.where(
        lax.broadcasted_iota(jnp.int32, (1, 128), 1) % 64 < 32, 1.0, -1.0)
    j32 = (lax.broadcasted_iota(jnp.int32, (32, 128), 0)
           == lax.broadcasted_iota(jnp.int32, (32, 128), 1) % 32
           ).astype(jnp.float32)

    def halfdiff(row):
        return row - pltpu.roll(row, 32, 1)

    r = [sel(k) for k in range(11)]

    emit((r[1] - r[0]) * sg)
    emit(halfdiff(r[0]))
    emit(halfdiff(r[1]))

    mx = jnp.maximum(r[2], r[3])
    mn = jnp.minimum(r[2], r[3])
    emit(r[4] - mx)
    emit(mn - r[4])
    emit(halfdiff(r[2]))
    emit(halfdiff(r[3]))
    emit(halfdiff(r[4]))

    for k, sgn in ((0, 1.0), (1, -1.0)):
        y = mm4(r[5 + 2 * k], r[6 + 2 * k])
        rc4 = jnp.dot(rel(k), j32, preferred_element_type=jnp.float32)
        d = pltpu.roll(y, 64, 1) - y
        emit((d - sgn * rc4) * sg128)
        emit(y - pltpu.roll(y, 96, 1))

    mx = jnp.maximum(r[9], r[10])
    mn = jnp.minimum(r[9], r[10])
    emit(mn - pltpu.roll(mx, 32, 1))
    emit(halfdiff(r[9]))
    emit(halfdiff(r[10]))

    total = jnp.sum(jnp.sqrt(s2))

    @pl.when(pl.program_id(0) == 0)
    def _():
        out_ref[...] = jnp.zeros((1, 1), jnp.float32)

    out_ref[...] += (total * (1.0 / _B)).reshape(1, 1)


def kernel(nf1, nf2, nf3, nf4, disjoint, classEmb, relEmb, Wr):
    idx_cols = [
        nf1[:, 0], nf1[:, 1],
        nf2[:, 0], nf2[:, 1], nf2[:, 2],
        nf3[:, 0], nf3[:, 2],
        nf4[:, 1], nf4[:, 2],
        disjoint[:, 0], disjoint[:, 1],
        nf3[:, 1], nf4[:, 0],
    ]
    idx_cols = [c.astype(jnp.int32) for c in idx_cols]
    cls_piece = [(c >> 14) & 1 for c in idx_cols[:_NCLS]]
    rel_piece = [(c >> 14) & 3 for c in idx_cols[_NCLS:]]
    cls_idx = jnp.concatenate(
        [((c >> 15) << 14) | (c & 16383) for c in idx_cols[:_NCLS]])
    rel_idx = jnp.concatenate(
        [((c >> 16) << 14) | (c & 16383) for c in idx_cols[_NCLS:]])
    piece_cols = cls_piece + rel_piece

    w4 = jnp.kron(jnp.eye(4, dtype=jnp.float32), Wr.astype(jnp.float32))
    msum = jnp.asarray(_route_matrix())

    cls128 = _pack_table(classEmb, 32768)
    crows = _make_sc_gather(_NCLS * _B, _CH_CLS)(cls_idx, cls128)
    rel128 = _pack_table(relEmb, 65536)
    rrows = _make_sc_gather(_NREL * _B, _CH_REL)(rel_idx, rel128)
    crows = crows.reshape(_NCLS, _B, 128)
    rrows = rrows.reshape(_NREL, _B, 128)

    out = pl.pallas_call(
        _tc_loss_body,
        grid=(_B // _BB,),
        in_specs=[pl.BlockSpec((_NCLS, _BB, 128), lambda i: (0, i, 0)),
                  pl.BlockSpec((_NREL, _BB, 128), lambda i: (0, i, 0))]
        + [pl.BlockSpec((_BB,), lambda i: (i,)) for _ in range(_NG)]
        + [pl.BlockSpec((128, 128), lambda i: (0, 0)),
           pl.BlockSpec((1216, 128), lambda i: (0, 0))],
        out_specs=pl.BlockSpec((1, 1), lambda i: (0, 0)),
        out_shape=jax.ShapeDtypeStruct((1, 1), jnp.float32),
    )(crows, rrows, *piece_cols, w4, msum)
    return out[0, 0]


# speedup vs baseline: 1.0563x; 1.0563x over previous
import functools

import jax
import jax.numpy as jnp
from jax import lax
from jax.experimental import pallas as pl
from jax.experimental.pallas import tpu as pltpu
from jax.experimental.pallas import tpu_sc as plsc

_DIM = 32
_B = 16384
_NC = 2
_NS = 16
_NW = _NC * _NS

_NCLS = 11
_NREL = 2
_NG = _NCLS + _NREL
_CLS_PER_W = _NCLS * _B // _NW
_REL_PER_W = _NREL * _B // _NW
_CH_CLS = 352
_CH_REL = 256
_NCH_CLS = _CLS_PER_W // _CH_CLS
_NCH_REL = _REL_PER_W // _CH_REL


def _tr_body(a_ref, out_ref):
    a = a_ref[...]
    d, w = a.shape
    blk = out_ref.shape[0]
    a2 = jnp.concatenate(
        [a[:, m * blk:(m + 1) * blk] for m in range(w // blk)], axis=0)
    eye = (lax.broadcasted_iota(jnp.int32, (128, 128), 0)
           == lax.broadcasted_iota(jnp.int32, (128, 128), 1)
           ).astype(jnp.float32)
    out_ref[...] = lax.dot_general(a2, eye, (((0,), (0,)), ((), ())),
                                   preferred_element_type=jnp.float32)


def _pack_table(emb, win):
    n, d = emb.shape
    v = emb.T
    blk = win * d // 128
    grid = (n + win - 1) // win
    return pl.pallas_call(
        _tr_body,
        grid=(grid,),
        in_specs=[pl.BlockSpec((d, win), lambda i: (0, i))],
        out_specs=pl.BlockSpec((blk, 128), lambda i: (i, 0)),
        out_shape=jax.ShapeDtypeStruct((grid * blk, 128), jnp.float32),
    )(v)


@functools.cache
def _make_sc_gather(total_rows, chunk):
    per_w = total_rows // _NW
    nch = per_w // chunk

    @functools.partial(
        pl.kernel,
        mesh=plsc.VectorSubcoreMesh(core_axis_name="c", subcore_axis_name="s"),
        out_type=jax.ShapeDtypeStruct((total_rows, 128), jnp.float32),
        scratch_types=[
            pltpu.VMEM((chunk,), jnp.int32),
            pltpu.VMEM((chunk,), jnp.int32),
            pltpu.VMEM((chunk, 128), jnp.float32),
            pltpu.VMEM((chunk, 128), jnp.float32),
            pltpu.SemaphoreType.DMA,
            pltpu.SemaphoreType.DMA,
        ],
    )
    def _sc_gather(idx_h, tab_h, out_h,
                   idx_a, idx_b, rows_a, rows_b, sem_a, sem_b):
        wid = lax.axis_index("s") * _NC + lax.axis_index("c")
        base = wid * per_w
        bufs = [(idx_a, rows_a, sem_a), (idx_b, rows_b, sem_b)]
        handles = [None, None]

        def start(k):
            i_v, r_v, s_v = bufs[k % 2]
            pltpu.sync_copy(idx_h.at[pl.ds(base + k * chunk, chunk)], i_v)
            handles[k % 2] = pltpu.async_copy(tab_h.at[i_v], r_v, s_v)

        start(0)
        for k in range(nch):
            if k + 1 < nch:
                start(k + 1)
            _, r_v, _ = bufs[k % 2]
            handles[k % 2].wait()
            pltpu.sync_copy(r_v, out_h.at[pl.ds(base + k * chunk, chunk)])

    return _sc_gather


_BB = 2048


def _tc_loss_body(crows_ref, rrows_ref, *rest):
    idx_refs = rest[:_NG]
    w4_ref, m_ref, out_ref = rest[_NG], rest[_NG + 1], rest[_NG + 2]
    d = _DIM

    def sel(k):
        g = crows_ref[k]
        b = ((idx_refs[k][...] & 1).reshape(_BB, 1)) != 0
        return jnp.where(b, g[:, 64:128], g[:, 0:64])

    def halves(k):
        row = sel(k)
        return row[:, :d], row[:, d:]

    def rel(k):
        g = rrows_ref[k]
        s = idx_refs[_NCLS + k][...]
        b0 = ((s & 1).reshape(_BB, 1)) != 0
        b1 = ((s & 2).reshape(_BB, 1)) != 0
        w01 = jnp.where(b0, g[:, 32:64], g[:, 0:32])
        w23 = jnp.where(b0, g[:, 96:128], g[:, 64:96])
        return jnp.where(b1, w23, w01)

    def mm4(r1, r2):
        y = jnp.dot(jnp.concatenate([r1, r2], axis=1), w4_ref[...],
                    preferred_element_type=jnp.float32)
        return y[:, 0:32], y[:, 32:64], y[:, 64:96], y[:, 96:128]

    terms = []

    cC, cO = halves(0)
    dC, dO = halves(1)
    terms += [dC - cC, cO - dO, cC - cO, dC - dO]

    cC, cO = halves(2)
    dC, dO = halves(3)
    eC, eO = halves(4)
    terms += [eC - jnp.maximum(cC, dC), jnp.minimum(cO, dO) - eO,
              cC - cO, dC - dO, eC - eO]

    aW, bW, pW, qW = mm4(sel(5), sel(6))
    rC = rel(0)
    cC, cO, dC, dO = aW + rC, bW + rC, pW, qW
    terms += [dC - cC, cO - dO, dC - dO, cC - cO]

    aW, bW, pW, qW = mm4(sel(7), sel(8))
    rC = rel(1)
    cC, cO, dC, dO = aW, bW, pW + rC, qW + rC
    terms += [dC - cC, cO - dO, dC - dO, cC - cO]

    cC, cO = halves(9)
    dC, dO = halves(10)
    terms += [jnp.minimum(cO, dO) - jnp.maximum(cC, dC), cC - cO, dC - dO]

    t = jnp.maximum(jnp.concatenate(terms, axis=1), 0.0)
    s2 = jnp.dot(t * t, m_ref[...], preferred_element_type=jnp.float32)
    total = jnp.sum(jnp.sqrt(s2))

    @pl.when(pl.program_id(0) == 0)
    def _():
        out_ref[...] = jnp.zeros((1, 1), jnp.float32)

    out_ref[...] += (total * (1.0 / _B)).reshape(1, 1)


def kernel(nf1, nf2, nf3, nf4, disjoint, classEmb, relEmb, Wr):
    idx_cols = [
        nf1[:, 0], nf1[:, 1],
        nf2[:, 0], nf2[:, 1], nf2[:, 2],
        nf3[:, 0], nf3[:, 2],
        nf4[:, 1], nf4[:, 2],
        disjoint[:, 0], disjoint[:, 1],
        nf3[:, 1], nf4[:, 0],
    ]
    idx_cols = [c.astype(jnp.int32) for c in idx_cols]
    cls_piece = [(c >> 14) & 1 for c in idx_cols[:_NCLS]]
    rel_piece = [(c >> 14) & 3 for c in idx_cols[_NCLS:]]
    cls_idx = jnp.concatenate(
        [((c >> 15) << 14) | (c & 16383) for c in idx_cols[:_NCLS]])
    rel_idx = jnp.concatenate(
        [((c >> 16) << 14) | (c & 16383) for c in idx_cols[_NCLS:]])
    piece_cols = cls_piece + rel_piece

    w4 = jnp.kron(jnp.eye(4, dtype=jnp.float32), Wr.astype(jnp.float32))
    msum = (jnp.arange(640)[:, None] // 32
            == jnp.arange(128)[None, :]).astype(jnp.float32)

    cls128 = _pack_table(classEmb, 32768)
    crows = _make_sc_gather(_NCLS * _B, _CH_CLS)(cls_idx, cls128)
    rel128 = _pack_table(relEmb, 65536)
    rrows = _make_sc_gather(_NREL * _B, _CH_REL)(rel_idx, rel128)
    crows = crows.reshape(_NCLS, _B, 128)
    rrows = rrows.reshape(_NREL, _B, 128)

    out = pl.pallas_call(
        _tc_loss_body,
        grid=(_B // _BB,),
        in_specs=[pl.BlockSpec((_NCLS, _BB, 128), lambda i: (0, i, 0)),
                  pl.BlockSpec((_NREL, _BB, 128), lambda i: (0, i, 0))]
        + [pl.BlockSpec((_BB,), lambda i: (i,)) for _ in range(_NG)]
        + [pl.BlockSpec((128, 128), lambda i: (0, 0)),
           pl.BlockSpec((640, 128), lambda i: (0, 0))],
        out_specs=pl.BlockSpec((1, 1), lambda i: (0, 0)),
        out_shape=jax.ShapeDtypeStruct((1, 1), jnp.float32),
    )(crows, rrows, *piece_cols, w4, msum)
    return out[0, 0]
